# baseline (speedup 1.0000x reference)
import functools

import jax
import jax.numpy as jnp
from jax import lax
from jax.experimental import pallas as pl
from jax.experimental.pallas import tpu as pltpu
from jax.experimental.pallas import tpu_sc as plsc

_NC = 2
_NS = 16
_NW = _NC * _NS
_LANES = 16


@functools.partial(jax.jit, static_argnums=(2, 3, 4))
def _sc_pos_add(x, pos_weight, B, T, D):
    t_per = T // _NW
    CH = 16
    n_ch = t_per // CH
    n_vec_row = D // _LANES
    UNROLL = 8
    n_steps = n_ch * B
    n_g = n_steps // 2

    mesh = plsc.VectorSubcoreMesh(core_axis_name="c", subcore_axis_name="s")

    @functools.partial(
        pl.kernel,
        mesh=mesh,
        out_type=jax.ShapeDtypeStruct((B, T, D), jnp.float32),
        scratch_types=[
            pltpu.VMEM((CH, D), jnp.float32),
            pltpu.VMEM((CH, D), jnp.float32),
            pltpu.VMEM((CH, D), jnp.float32),
            pltpu.VMEM((CH, D), jnp.float32),
            pltpu.VMEM((CH, D), jnp.float32),
            pltpu.SemaphoreType.DMA,
            pltpu.SemaphoreType.DMA,
            pltpu.SemaphoreType.DMA,
            pltpu.SemaphoreType.DMA,
        ],
    )
    def k(x_hbm, pos_hbm, out_hbm, pos_v, x0, x1, o0, o1, sx0, sx1, so0, so1):
        w = lax.axis_index("s") * _NC + lax.axis_index("c")
        t0 = w * t_per
        xbufs, obufs = (x0, x1), (o0, o1)
        sxs, sos = (sx0, sx1), (so0, so1)

        def coords(s):
            c = s // B
            b = s - c * B
            return b, t0 + c * CH

        b0, r0 = coords(0)
        b1, r1 = coords(1)
        pltpu.async_copy(x_hbm.at[b0, pl.ds(r0, CH)], x0, sx0)
        pltpu.async_copy(x_hbm.at[b1, pl.ds(r1, CH)], x1, sx1)

        def g_body(g, _):
            for u in (0, 1):
                s = g * 2 + u
                b, row0 = coords(s)
                xv, ov, sx, so = xbufs[u], obufs[u], sxs[u], sos[u]

                @pl.when(b == 0)
                def _():
                    pltpu.sync_copy(pos_hbm.at[pl.ds(row0, CH)], pos_v)

                pltpu.make_async_copy(x_hbm.at[0, pl.ds(0, CH)], xv, sx).wait()

                @pl.when(g > 0)
                def _():
                    pltpu.make_async_copy(
                        ov, out_hbm.at[0, pl.ds(0, CH)], so
                    ).wait()

                for r in range(CH):
                    for j in range(n_vec_row):
                        off = j * _LANES
                        ov[r, pl.ds(off, _LANES)] = (
                            xv[r, pl.ds(off, _LANES)]
                            + pos_v[r, pl.ds(off, _LANES)]
                        )

                pltpu.async_copy(ov, out_hbm.at[b, pl.ds(row0, CH)], so)

                @pl.when(g < n_g - 1)
                def _():
                    b2, row2 = coords(s + 2)
                    pltpu.async_copy(x_hbm.at[b2, pl.ds(row2, CH)], xv, sx)
            return 0

        lax.fori_loop(0, n_g, g_body, 0, unroll=False)
        pltpu.make_async_copy(o0, out_hbm.at[0, pl.ds(0, CH)], so0).wait()
        pltpu.make_async_copy(o1, out_hbm.at[0, pl.ds(0, CH)], so1).wait()

    return k(x, pos_weight)


def kernel(x, pos_weight):
    B, T, D = x.shape
    return _sc_pos_add(x, pos_weight, B, T, D)


# speedup vs baseline: 2.1889x; 2.1889x over previous
import functools

import jax
import jax.numpy as jnp
from jax import lax
from jax.experimental import pallas as pl
from jax.experimental.pallas import tpu as pltpu
from jax.experimental.pallas import tpu_sc as plsc

_NC = 2
_NS = 16
_NW = _NC * _NS
_LANES = 16


@functools.partial(jax.jit, static_argnums=(2, 3, 4, 5))
def _sc_pos_add(x, pos_weight, b_lo, B, T, D):
    nb = B - b_lo
    t_per = T // _NW
    CH = 16
    n_ch = t_per // CH
    n_vec_row = D // _LANES
    UNROLL = 8
    n_steps = n_ch * nb
    n_g = n_steps // 2

    mesh = plsc.VectorSubcoreMesh(core_axis_name="c", subcore_axis_name="s")

    @functools.partial(
        pl.kernel,
        mesh=mesh,
        out_type=jax.ShapeDtypeStruct((nb, T, D), jnp.float32),
        scratch_types=[
            pltpu.VMEM((CH, D), jnp.float32),
            pltpu.VMEM((CH, D), jnp.float32),
            pltpu.VMEM((CH, D), jnp.float32),
            pltpu.VMEM((CH, D), jnp.float32),
            pltpu.VMEM((CH, D), jnp.float32),
            pltpu.VMEM((CH, D), jnp.float32),
            pltpu.SemaphoreType.DMA,
            pltpu.SemaphoreType.DMA,
            pltpu.SemaphoreType.DMA,
            pltpu.SemaphoreType.DMA,
            pltpu.SemaphoreType.DMA,
            pltpu.SemaphoreType.DMA,
        ],
    )
    def k(x_hbm, pos_hbm, out_hbm,
          x0, x1, p0, p1, o0, o1, sx0, sx1, sp0, sp1, so0, so1):
        w = lax.axis_index("s") * _NC + lax.axis_index("c")
        t0 = w * t_per
        xbufs, pbufs, obufs = (x0, x1), (p0, p1), (o0, o1)
        sxs, sps, sos = (sx0, sx1), (sp0, sp1), (so0, so1)

        def coords(s):
            c = s // nb
            b = s - c * nb
            return b, t0 + c * CH

        for u in (0, 1):
            b, r = coords(u)
            pltpu.async_copy(x_hbm.at[b_lo + b, pl.ds(r, CH)], xbufs[u], sxs[u])
            pltpu.async_copy(pos_hbm.at[pl.ds(r, CH)], pbufs[u], sps[u])

        def g_body(g, _):
            for u in (0, 1):
                s = g * 2 + u
                b, row0 = coords(s)
                xv, pv, ov = xbufs[u], pbufs[u], obufs[u]
                sx, sp, so = sxs[u], sps[u], sos[u]

                pltpu.make_async_copy(x_hbm.at[0, pl.ds(0, CH)], xv, sx).wait()
                pltpu.make_async_copy(pos_hbm.at[pl.ds(0, CH)], pv, sp).wait()

                @pl.when(g > 0)
                def _():
                    pltpu.make_async_copy(
                        ov, out_hbm.at[0, pl.ds(0, CH)], so
                    ).wait()

                def add_body(j, _):
                    base = j * (_LANES * UNROLL)
                    for r in range(CH):
                        for uu in range(UNROLL):
                            off = base + uu * _LANES
                            ov[r, pl.ds(off, _LANES)] = (
                                xv[r, pl.ds(off, _LANES)]
                                + pv[r, pl.ds(off, _LANES)]
                            )
                    return 0

                lax.fori_loop(
                    0, n_vec_row // UNROLL, add_body, 0, unroll=False
                )

                pltpu.async_copy(ov, out_hbm.at[b, pl.ds(row0, CH)], so)

                @pl.when(g < n_g - 1)
                def _():
                    b2, row2 = coords(s + 2)
                    pltpu.async_copy(
                        x_hbm.at[b_lo + b2, pl.ds(row2, CH)], xv, sx
                    )
                    pltpu.async_copy(pos_hbm.at[pl.ds(row2, CH)], pv, sp)
            return 0

        lax.fori_loop(0, n_g, g_body, 0, unroll=False)
        pltpu.make_async_copy(o0, out_hbm.at[0, pl.ds(0, CH)], so0).wait()
        pltpu.make_async_copy(o1, out_hbm.at[0, pl.ds(0, CH)], so1).wait()

    return k(x, pos_weight)


def _tc_add_body(x_ref, p_ref, o_ref):
    o_ref[...] = x_ref[...] + p_ref[...]


def _tc_pos_add(x, pos_weight, nb, BLK):
    B, T, D = x.shape
    return pl.pallas_call(
        _tc_add_body,
        grid=(T // BLK,),
        in_specs=[
            pl.BlockSpec((nb, BLK, D), lambda t: (0, t, 0)),
            pl.BlockSpec((BLK, D), lambda t: (t, 0)),
        ],
        out_specs=pl.BlockSpec((nb, BLK, D), lambda t: (0, t, 0)),
        out_shape=jax.ShapeDtypeStruct((nb, T, D), x.dtype),
    )(x, pos_weight)


def kernel(x, pos_weight):
    B, T, D = x.shape
    SC_NB = 1
    tc_out = _tc_pos_add(x, pos_weight, B - SC_NB, 512)
    sc_out = _sc_pos_add(x, pos_weight, B - SC_NB, B, T, D)
    return jnp.concatenate([tc_out, sc_out], axis=0)


# speedup vs baseline: 3.6644x; 1.6741x over previous
import functools

import jax
import jax.numpy as jnp
from jax import lax
from jax.experimental import pallas as pl
from jax.experimental.pallas import tpu as pltpu
from jax.experimental.pallas import tpu_sc as plsc

_NC = 2
_NS = 16
_NW = _NC * _NS
_LANES = 16


@functools.partial(jax.jit, static_argnums=(2, 3))
def _sc_pos_add(x, pos_weight, t_lo, ts):
    B, T, D = x.shape
    t_per = ts // _NW
    CH = 16
    n_ch = t_per // CH
    n_vec_row = D // _LANES
    UNROLL = 8
    n_steps = n_ch * B
    n_g = n_steps // 2

    mesh = plsc.VectorSubcoreMesh(core_axis_name="c", subcore_axis_name="s")

    @functools.partial(
        pl.kernel,
        mesh=mesh,
        out_type=jax.ShapeDtypeStruct((B, ts, D), jnp.float32),
        scratch_types=[
            pltpu.VMEM((CH, D), jnp.float32),
            pltpu.VMEM((CH, D), jnp.float32),
            pltpu.VMEM((CH, D), jnp.float32),
            pltpu.VMEM((CH, D), jnp.float32),
            pltpu.VMEM((CH, D), jnp.float32),
            pltpu.SemaphoreType.DMA,
            pltpu.SemaphoreType.DMA,
            pltpu.SemaphoreType.DMA,
            pltpu.SemaphoreType.DMA,
        ],
    )
    def k(x_hbm, pos_hbm, out_hbm, pos_v, x0, x1, o0, o1, sx0, sx1, so0, so1):
        w = lax.axis_index("s") * _NC + lax.axis_index("c")
        t0 = w * t_per
        xbufs, obufs = (x0, x1), (o0, o1)
        sxs, sos = (sx0, sx1), (so0, so1)

        def coords(s):
            c = s // B
            b = s - c * B
            return b, t0 + c * CH

        for u in (0, 1):
            b, r = coords(u)
            pltpu.async_copy(x_hbm.at[b, pl.ds(t_lo + r, CH)], xbufs[u], sxs[u])

        def g_body(g, _):
            for u in (0, 1):
                s = g * 2 + u
                b, row0 = coords(s)
                xv, ov = xbufs[u], obufs[u]
                sx, so = sxs[u], sos[u]

                @pl.when(b == 0)
                def _():
                    pltpu.sync_copy(
                        pos_hbm.at[pl.ds(t_lo + row0, CH)], pos_v
                    )

                pltpu.make_async_copy(x_hbm.at[0, pl.ds(0, CH)], xv, sx).wait()

                @pl.when(g > 0)
                def _():
                    pltpu.make_async_copy(
                        ov, out_hbm.at[0, pl.ds(0, CH)], so
                    ).wait()

                def add_body(j, _):
                    base = j * (_LANES * UNROLL)
                    for r in range(CH):
                        for uu in range(UNROLL):
                            off = base + uu * _LANES
                            ov[r, pl.ds(off, _LANES)] = (
                                xv[r, pl.ds(off, _LANES)]
                                + pos_v[r, pl.ds(off, _LANES)]
                            )
                    return 0

                lax.fori_loop(
                    0, n_vec_row // UNROLL, add_body, 0, unroll=False
                )

                pltpu.async_copy(ov, out_hbm.at[b, pl.ds(row0, CH)], so)

                @pl.when(g < n_g - 1)
                def _():
                    b2, row2 = coords(s + 2)
                    pltpu.async_copy(
                        x_hbm.at[b2, pl.ds(t_lo + row2, CH)], xv, sx
                    )
            return 0

        lax.fori_loop(0, n_g, g_body, 0, unroll=False)
        pltpu.make_async_copy(o0, out_hbm.at[0, pl.ds(0, CH)], so0).wait()
        pltpu.make_async_copy(o1, out_hbm.at[0, pl.ds(0, CH)], so1).wait()

    return k(x, pos_weight)


def _tc_add_body(x_ref, p_ref, o_ref):
    o_ref[...] = x_ref[...] + p_ref[...]


def _tc_pos_add(x, pos_weight, n_t, BLK):
    B, T, D = x.shape
    return pl.pallas_call(
        _tc_add_body,
        grid=(n_t // BLK,),
        in_specs=[
            pl.BlockSpec((B, BLK, D), lambda t: (0, t, 0)),
            pl.BlockSpec((BLK, D), lambda t: (t, 0)),
        ],
        out_specs=pl.BlockSpec((B, BLK, D), lambda t: (0, t, 0)),
        out_shape=jax.ShapeDtypeStruct((B, T, D), x.dtype),
    )(x, pos_weight)


def kernel(x, pos_weight):
    B, T, D = x.shape
    TS = T // 8
    tc_out = _tc_pos_add(x, pos_weight, T - TS, 512)
    sc_out = _sc_pos_add(x, pos_weight, T - TS, TS)
    return lax.dynamic_update_slice(tc_out, sc_out, (0, T - TS, 0))


# speedup vs baseline: 3.8233x; 1.0434x over previous
import functools

import jax
import jax.numpy as jnp
from jax import lax
from jax.experimental import pallas as pl
from jax.experimental.pallas import tpu as pltpu
from jax.experimental.pallas import tpu_sc as plsc

_NC = 2
_NS = 16
_NW = _NC * _NS
_LANES = 16


@functools.partial(jax.jit, static_argnums=(2, 3))
def _sc_pos_add(x, pos_weight, t_lo, ts):
    B, T, D = x.shape
    t_per = ts // _NW
    CH = 16
    n_ch = t_per // CH
    n_vec_row = D // _LANES
    UNROLL = 8
    n_steps = n_ch * B
    n_g = n_steps // 2

    mesh = plsc.VectorSubcoreMesh(core_axis_name="c", subcore_axis_name="s")

    @functools.partial(
        pl.kernel,
        mesh=mesh,
        out_type=jax.ShapeDtypeStruct((B, ts, D), jnp.float32),
        scratch_types=[
            pltpu.VMEM((CH, D), jnp.float32),
            pltpu.VMEM((CH, D), jnp.float32),
            pltpu.VMEM((CH, D), jnp.float32),
            pltpu.VMEM((CH, D), jnp.float32),
            pltpu.VMEM((CH, D), jnp.float32),
            pltpu.SemaphoreType.DMA,
            pltpu.SemaphoreType.DMA,
            pltpu.SemaphoreType.DMA,
            pltpu.SemaphoreType.DMA,
        ],
    )
    def k(x_hbm, pos_hbm, out_hbm, pos_v, x0, x1, o0, o1, sx0, sx1, so0, so1):
        w = lax.axis_index("s") * _NC + lax.axis_index("c")
        t0 = w * t_per
        xbufs, obufs = (x0, x1), (o0, o1)
        sxs, sos = (sx0, sx1), (so0, so1)

        def coords(s):
            c = s // B
            b = s - c * B
            return b, t0 + c * CH

        for u in (0, 1):
            b, r = coords(u)
            pltpu.async_copy(x_hbm.at[b, pl.ds(t_lo + r, CH)], xbufs[u], sxs[u])

        def g_body(g, _):
            for u in (0, 1):
                s = g * 2 + u
                b, row0 = coords(s)
                xv, ov = xbufs[u], obufs[u]
                sx, so = sxs[u], sos[u]

                @pl.when(b == 0)
                def _():
                    pltpu.sync_copy(
                        pos_hbm.at[pl.ds(t_lo + row0, CH)], pos_v
                    )

                pltpu.make_async_copy(x_hbm.at[0, pl.ds(0, CH)], xv, sx).wait()

                @pl.when(g > 0)
                def _():
                    pltpu.make_async_copy(
                        ov, out_hbm.at[0, pl.ds(0, CH)], so
                    ).wait()

                def add_body(j, _):
                    base = j * (_LANES * UNROLL)
                    for r in range(CH):
                        for uu in range(UNROLL):
                            off = base + uu * _LANES
                            ov[r, pl.ds(off, _LANES)] = (
                                xv[r, pl.ds(off, _LANES)]
                                + pos_v[r, pl.ds(off, _LANES)]
                            )
                    return 0

                lax.fori_loop(
                    0, n_vec_row // UNROLL, add_body, 0, unroll=False
                )

                pltpu.async_copy(ov, out_hbm.at[b, pl.ds(row0, CH)], so)

                @pl.when(g < n_g - 1)
                def _():
                    b2, row2 = coords(s + 2)
                    pltpu.async_copy(
                        x_hbm.at[b2, pl.ds(t_lo + row2, CH)], xv, sx
                    )
            return 0

        lax.fori_loop(0, n_g, g_body, 0, unroll=False)
        pltpu.make_async_copy(o0, out_hbm.at[0, pl.ds(0, CH)], so0).wait()
        pltpu.make_async_copy(o1, out_hbm.at[0, pl.ds(0, CH)], so1).wait()

    return k(x, pos_weight)


def _tc_add_body(x_ref, p_ref, o_ref):
    o_ref[...] = x_ref[...] + p_ref[...]


def _tc_pos_add(x, pos_weight, n_t, BLK):
    B, T, D = x.shape
    return pl.pallas_call(
        _tc_add_body,
        grid=(n_t // BLK,),
        in_specs=[
            pl.BlockSpec((B, BLK, D), lambda t: (0, t, 0)),
            pl.BlockSpec((BLK, D), lambda t: (t, 0)),
        ],
        out_specs=pl.BlockSpec((B, BLK, D), lambda t: (0, t, 0)),
        out_shape=jax.ShapeDtypeStruct((B, T, D), x.dtype),
    )(x, pos_weight)


def kernel(x, pos_weight):
    B, T, D = x.shape
    TS = T // 16
    tc_out = _tc_pos_add(x, pos_weight, T - TS, 512)
    sc_out = _sc_pos_add(x, pos_weight, T - TS, TS)
    return lax.dynamic_update_slice(tc_out, sc_out, (0, T - TS, 0))


# speedup vs baseline: 4.0540x; 1.0604x over previous
import functools

import jax
import jax.numpy as jnp
from jax import lax
from jax.experimental import pallas as pl
from jax.experimental.pallas import tpu as pltpu
from jax.experimental.pallas import tpu_sc as plsc

_NC = 2
_NS = 16
_NW = _NC * _NS
_LANES = 16


@functools.partial(jax.jit, static_argnums=(2, 3))
def _sc_pos_add(x, pos_weight, t_lo, ts):
    B, T, D = x.shape
    t_per = ts // _NW
    CH = min(16, t_per)
    n_ch = t_per // CH
    n_vec_row = D // _LANES
    UNROLL = 8
    n_steps = n_ch * B
    n_g = n_steps // 2

    mesh = plsc.VectorSubcoreMesh(core_axis_name="c", subcore_axis_name="s")

    @functools.partial(
        pl.kernel,
        mesh=mesh,
        out_type=jax.ShapeDtypeStruct((B, ts, D), jnp.float32),
        scratch_types=[
            pltpu.VMEM((CH, D), jnp.float32),
            pltpu.VMEM((CH, D), jnp.float32),
            pltpu.VMEM((CH, D), jnp.float32),
            pltpu.VMEM((CH, D), jnp.float32),
            pltpu.VMEM((CH, D), jnp.float32),
            pltpu.SemaphoreType.DMA,
            pltpu.SemaphoreType.DMA,
            pltpu.SemaphoreType.DMA,
            pltpu.SemaphoreType.DMA,
        ],
    )
    def k(x_hbm, pos_hbm, out_hbm, pos_v, x0, x1, o0, o1, sx0, sx1, so0, so1):
        w = lax.axis_index("s") * _NC + lax.axis_index("c")
        t0 = w * t_per
        xbufs, obufs = (x0, x1), (o0, o1)
        sxs, sos = (sx0, sx1), (so0, so1)

        def coords(s):
            c = s // B
            b = s - c * B
            return b, t0 + c * CH

        for u in (0, 1):
            b, r = coords(u)
            pltpu.async_copy(x_hbm.at[b, pl.ds(t_lo + r, CH)], xbufs[u], sxs[u])

        def g_body(g, _):
            for u in (0, 1):
                s = g * 2 + u
                b, row0 = coords(s)
                xv, ov = xbufs[u], obufs[u]
                sx, so = sxs[u], sos[u]

                @pl.when(b == 0)
                def _():
                    pltpu.sync_copy(
                        pos_hbm.at[pl.ds(t_lo + row0, CH)], pos_v
                    )

                pltpu.make_async_copy(x_hbm.at[0, pl.ds(0, CH)], xv, sx).wait()

                @pl.when(g > 0)
                def _():
                    pltpu.make_async_copy(
                        ov, out_hbm.at[0, pl.ds(0, CH)], so
                    ).wait()

                def add_body(j, _):
                    base = j * (_LANES * UNROLL)
                    for r in range(CH):
                        for uu in range(UNROLL):
                            off = base + uu * _LANES
                            ov[r, pl.ds(off, _LANES)] = (
                                xv[r, pl.ds(off, _LANES)]
                                + pos_v[r, pl.ds(off, _LANES)]
                            )
                    return 0

                lax.fori_loop(
                    0, n_vec_row // UNROLL, add_body, 0, unroll=False
                )

                pltpu.async_copy(ov, out_hbm.at[b, pl.ds(row0, CH)], so)

                @pl.when(g < n_g - 1)
                def _():
                    b2, row2 = coords(s + 2)
                    pltpu.async_copy(
                        x_hbm.at[b2, pl.ds(t_lo + row2, CH)], xv, sx
                    )
            return 0

        lax.fori_loop(0, n_g, g_body, 0, unroll=False)
        pltpu.make_async_copy(o0, out_hbm.at[0, pl.ds(0, CH)], so0).wait()
        pltpu.make_async_copy(o1, out_hbm.at[0, pl.ds(0, CH)], so1).wait()

    return k(x, pos_weight)


def _tc_add_body(x_ref, p_ref, o_ref):
    o_ref[...] = x_ref[...] + p_ref[...]


def _tc_pos_add(x, pos_weight, n_t, BLK):
    B, T, D = x.shape
    return pl.pallas_call(
        _tc_add_body,
        grid=(n_t // BLK,),
        in_specs=[
            pl.BlockSpec((B, BLK, D), lambda t: (0, t, 0)),
            pl.BlockSpec((BLK, D), lambda t: (t, 0)),
        ],
        out_specs=pl.BlockSpec((B, BLK, D), lambda t: (0, t, 0)),
        out_shape=jax.ShapeDtypeStruct((B, T, D), x.dtype),
    )(x, pos_weight)


def kernel(x, pos_weight):
    B, T, D = x.shape
    TS = T // 32
    tc_out = _tc_pos_add(x, pos_weight, T - TS, 512)
    sc_out = _sc_pos_add(x, pos_weight, T - TS, TS)
    return lax.dynamic_update_slice(tc_out, sc_out, (0, T - TS, 0))
